# baseline (speedup 1.0000x reference)
import functools

import jax
import jax.numpy as jnp
from jax import lax
from jax.experimental import pallas as pl
from jax.experimental.pallas import tpu as pltpu
from jax.experimental.pallas import tpu_sc as plsc

BATCH = 16
NUM_SEG = 64
SEG_LEN = 128
EMB = 128

NC = 2
NS = 16
NW = NC * NS
ECH = EMB // 16

SC_BATCH = 4
TC_BATCH = BATCH - SC_BATCH
NBUF = 4
RU = 2


def _sc_run(x3, seg_table, pos_table):
    slabs = SC_BATCH * NUM_SEG
    spw = slabs // NW
    mesh = plsc.VectorSubcoreMesh(core_axis_name="c", subcore_axis_name="s")

    @functools.partial(
        pl.kernel,
        out_type=jax.ShapeDtypeStruct((slabs, SEG_LEN, EMB), jnp.float32),
        mesh=mesh,
        scratch_types=[
            pltpu.VMEM((NUM_SEG, EMB), jnp.float32),
            pltpu.VMEM((SEG_LEN, EMB), jnp.float32),
            [pltpu.VMEM((SEG_LEN, EMB), jnp.float32)] * NBUF,
            [pltpu.SemaphoreType.DMA] * NBUF,
            [pltpu.SemaphoreType.DMA] * NBUF,
            pltpu.SemaphoreType.DMA,
        ],
    )
    def run(x_hbm, seg_hbm, pos_hbm, out_hbm, seg_v, pos_v, bufs, in_sems,
            out_sems, tab_sem):
        c = lax.axis_index("c")
        s = lax.axis_index("s")
        wid = s * NC + c
        base = wid * spw

        def start_in(i, b):
            return pltpu.async_copy(x_hbm.at[base + i], bufs[b], in_sems[b])

        seg_h = pltpu.async_copy(seg_hbm, seg_v, tab_sem)
        pos_h = pltpu.async_copy(pos_hbm, pos_v, tab_sem)

        in_h = {}
        out_h = {}
        for i in range(NBUF - 1):
            in_h[i] = start_in(i, i)
        seg_h.wait()
        pos_h.wait()

        for i in range(spw):
            b = i % NBUF
            in_h.pop(i).wait()
            slab = base + i
            n = lax.rem(slab, NUM_SEG)
            buf = bufs[b]
            segv = tuple(seg_v[n, pl.ds(e * 16, 16)] for e in range(ECH))

            def row(r, segc, buf=buf):
                for u in range(RU):
                    l = r * RU + u
                    for e in range(ECH):
                        xv = buf[l, pl.ds(e * 16, 16)]
                        pv = pos_v[l, pl.ds(e * 16, 16)]
                        buf[l, pl.ds(e * 16, 16)] = xv + pv + segc[e]
                return segc

            lax.fori_loop(0, SEG_LEN // RU, row, segv)
            out_h[i] = pltpu.async_copy(buf, out_hbm.at[slab], out_sems[b])

            nxt = i + NBUF - 1
            if nxt < spw:
                bn = nxt % NBUF
                prev = nxt - NBUF
                if prev >= 0:
                    out_h.pop(prev).wait()
                in_h[nxt] = start_in(nxt, bn)

        for i in sorted(out_h):
            out_h.pop(i).wait()

    return run(x3, seg_table, pos_table)


def _tc_body(x_ref, seg_ref, pos_ref, out_ref):
    x = x_ref[...]
    seg = seg_ref[...]
    pos = pos_ref[...]
    out_ref[...] = x + seg[None, :, None, :] + pos[None, None, :, :]


def _tc_run(x, seg_table, pos_table):
    return pl.pallas_call(
        _tc_body,
        grid=(TC_BATCH,),
        in_specs=[
            pl.BlockSpec((1, NUM_SEG, SEG_LEN, EMB),
                         lambda b: (b + SC_BATCH, 0, 0, 0)),
            pl.BlockSpec((NUM_SEG, EMB), lambda b: (0, 0)),
            pl.BlockSpec((SEG_LEN, EMB), lambda b: (0, 0)),
        ],
        out_specs=pl.BlockSpec((1, NUM_SEG, SEG_LEN, EMB),
                               lambda b: (b + SC_BATCH, 0, 0, 0)),
        out_shape=jax.ShapeDtypeStruct(
            (BATCH, NUM_SEG, SEG_LEN, EMB), x.dtype),
        compiler_params=pltpu.CompilerParams(
            dimension_semantics=("arbitrary",),
        ),
    )(x, seg_table, pos_table)


def kernel(x, seg_table, pos_table):
    x3 = x.reshape(BATCH * NUM_SEG, SEG_LEN, EMB)
    sc_out = _sc_run(x3, seg_table, pos_table)
    tc_out = _tc_run(x, seg_table, pos_table)
    return lax.dynamic_update_slice(
        tc_out, sc_out.reshape(SC_BATCH, NUM_SEG, SEG_LEN, EMB), (0, 0, 0, 0))


# speedup vs baseline: 1.0700x; 1.0700x over previous
import functools

import jax
import jax.numpy as jnp
from jax import lax
from jax.experimental import pallas as pl
from jax.experimental.pallas import tpu as pltpu
from jax.experimental.pallas import tpu_sc as plsc

BATCH = 16
NUM_SEG = 64
SEG_LEN = 128
EMB = 128

NC = 2
NS = 16
NW = NC * NS
ECH = EMB // 16

SC_BATCH = 2
TC_BATCH = BATCH - SC_BATCH
NBUF = 4
RU = 2


def _sc_run(x3, seg_table, pos_table):
    slabs = SC_BATCH * NUM_SEG
    spw = slabs // NW
    mesh = plsc.VectorSubcoreMesh(core_axis_name="c", subcore_axis_name="s")

    @functools.partial(
        pl.kernel,
        out_type=jax.ShapeDtypeStruct((slabs, SEG_LEN, EMB), jnp.float32),
        mesh=mesh,
        scratch_types=[
            pltpu.VMEM((NUM_SEG, EMB), jnp.float32),
            pltpu.VMEM((SEG_LEN, EMB), jnp.float32),
            [pltpu.VMEM((SEG_LEN, EMB), jnp.float32)] * NBUF,
            [pltpu.SemaphoreType.DMA] * NBUF,
            [pltpu.SemaphoreType.DMA] * NBUF,
            pltpu.SemaphoreType.DMA,
        ],
    )
    def run(x_hbm, seg_hbm, pos_hbm, out_hbm, seg_v, pos_v, bufs, in_sems,
            out_sems, tab_sem):
        c = lax.axis_index("c")
        s = lax.axis_index("s")
        wid = s * NC + c
        base = wid * spw

        def start_in(i, b):
            return pltpu.async_copy(x_hbm.at[base + i], bufs[b], in_sems[b])

        seg_h = pltpu.async_copy(seg_hbm, seg_v, tab_sem)
        pos_h = pltpu.async_copy(pos_hbm, pos_v, tab_sem)

        in_h = {}
        out_h = {}
        for i in range(NBUF - 1):
            in_h[i] = start_in(i, i)
        seg_h.wait()
        pos_h.wait()

        for i in range(spw):
            b = i % NBUF
            in_h.pop(i).wait()
            slab = base + i
            n = lax.rem(slab, NUM_SEG)
            buf = bufs[b]
            segv = tuple(seg_v[n, pl.ds(e * 16, 16)] for e in range(ECH))

            def row(r, segc, buf=buf):
                for u in range(RU):
                    l = r * RU + u
                    for e in range(ECH):
                        xv = buf[l, pl.ds(e * 16, 16)]
                        pv = pos_v[l, pl.ds(e * 16, 16)]
                        buf[l, pl.ds(e * 16, 16)] = xv + pv + segc[e]
                return segc

            lax.fori_loop(0, SEG_LEN // RU, row, segv)
            out_h[i] = pltpu.async_copy(buf, out_hbm.at[slab], out_sems[b])

            nxt = i + NBUF - 1
            if nxt < spw:
                bn = nxt % NBUF
                prev = nxt - NBUF
                if prev >= 0:
                    out_h.pop(prev).wait()
                in_h[nxt] = start_in(nxt, bn)

        for i in sorted(out_h):
            out_h.pop(i).wait()

    return run(x3, seg_table, pos_table)


def _tc_body(x_ref, seg_ref, pos_ref, out_ref):
    x = x_ref[...]
    seg = seg_ref[...]
    pos = pos_ref[...]
    out_ref[...] = x + seg[None, :, None, :] + pos[None, None, :, :]


def _tc_run(x, seg_table, pos_table):
    return pl.pallas_call(
        _tc_body,
        grid=(TC_BATCH,),
        in_specs=[
            pl.BlockSpec((1, NUM_SEG, SEG_LEN, EMB),
                         lambda b: (b + SC_BATCH, 0, 0, 0)),
            pl.BlockSpec((NUM_SEG, EMB), lambda b: (0, 0)),
            pl.BlockSpec((SEG_LEN, EMB), lambda b: (0, 0)),
        ],
        out_specs=pl.BlockSpec((1, NUM_SEG, SEG_LEN, EMB),
                               lambda b: (b + SC_BATCH, 0, 0, 0)),
        out_shape=jax.ShapeDtypeStruct(
            (BATCH, NUM_SEG, SEG_LEN, EMB), x.dtype),
        compiler_params=pltpu.CompilerParams(
            dimension_semantics=("arbitrary",),
        ),
    )(x, seg_table, pos_table)


def kernel(x, seg_table, pos_table):
    x3 = x.reshape(BATCH * NUM_SEG, SEG_LEN, EMB)
    sc_out = _sc_run(x3, seg_table, pos_table)
    tc_out = _tc_run(x, seg_table, pos_table)
    return lax.dynamic_update_slice(
        tc_out, sc_out.reshape(SC_BATCH, NUM_SEG, SEG_LEN, EMB), (0, 0, 0, 0))


# speedup vs baseline: 1.1216x; 1.0482x over previous
import functools

import jax
import jax.numpy as jnp
from jax import lax
from jax.experimental import pallas as pl
from jax.experimental.pallas import tpu as pltpu
from jax.experimental.pallas import tpu_sc as plsc

BATCH = 16
NUM_SEG = 64
SEG_LEN = 128
EMB = 128

NC = 2
NS = 16
NW = NC * NS
ECH = EMB // 16

SC_BATCH = 1
TC_BATCH = BATCH - SC_BATCH
NBUF = 2
RU = 2


def _sc_run(x3, seg_table, pos_table):
    slabs = SC_BATCH * NUM_SEG
    spw = slabs // NW
    mesh = plsc.VectorSubcoreMesh(core_axis_name="c", subcore_axis_name="s")

    @functools.partial(
        pl.kernel,
        out_type=jax.ShapeDtypeStruct((slabs, SEG_LEN, EMB), jnp.float32),
        mesh=mesh,
        scratch_types=[
            pltpu.VMEM((NUM_SEG, EMB), jnp.float32),
            pltpu.VMEM((SEG_LEN, EMB), jnp.float32),
            [pltpu.VMEM((SEG_LEN, EMB), jnp.float32)] * NBUF,
            [pltpu.SemaphoreType.DMA] * NBUF,
            [pltpu.SemaphoreType.DMA] * NBUF,
            pltpu.SemaphoreType.DMA,
        ],
    )
    def run(x_hbm, seg_hbm, pos_hbm, out_hbm, seg_v, pos_v, bufs, in_sems,
            out_sems, tab_sem):
        c = lax.axis_index("c")
        s = lax.axis_index("s")
        wid = s * NC + c
        base = wid * spw

        def start_in(i, b):
            return pltpu.async_copy(x_hbm.at[base + i], bufs[b], in_sems[b])

        seg_h = pltpu.async_copy(seg_hbm, seg_v, tab_sem)
        pos_h = pltpu.async_copy(pos_hbm, pos_v, tab_sem)

        in_h = {}
        out_h = {}
        for i in range(min(NBUF - 1, spw)):
            in_h[i] = start_in(i, i)
        seg_h.wait()
        pos_h.wait()

        for i in range(spw):
            b = i % NBUF
            in_h.pop(i).wait()
            slab = base + i
            n = lax.rem(slab, NUM_SEG)
            buf = bufs[b]
            segv = tuple(seg_v[n, pl.ds(e * 16, 16)] for e in range(ECH))

            def row(r, segc, buf=buf):
                for u in range(RU):
                    l = r * RU + u
                    for e in range(ECH):
                        xv = buf[l, pl.ds(e * 16, 16)]
                        pv = pos_v[l, pl.ds(e * 16, 16)]
                        buf[l, pl.ds(e * 16, 16)] = xv + pv + segc[e]
                return segc

            lax.fori_loop(0, SEG_LEN // RU, row, segv)
            out_h[i] = pltpu.async_copy(buf, out_hbm.at[slab], out_sems[b])

            nxt = i + NBUF - 1
            if nxt < spw:
                bn = nxt % NBUF
                prev = nxt - NBUF
                if prev >= 0:
                    out_h.pop(prev).wait()
                in_h[nxt] = start_in(nxt, bn)

        for i in sorted(out_h):
            out_h.pop(i).wait()

    return run(x3, seg_table, pos_table)


def _tc_body(x_ref, seg_ref, pos_ref, out_ref):
    x = x_ref[...]
    seg = seg_ref[...]
    pos = pos_ref[...]
    out_ref[...] = x + seg[None, :, None, :] + pos[None, None, :, :]


def _tc_run(x, seg_table, pos_table):
    return pl.pallas_call(
        _tc_body,
        grid=(TC_BATCH,),
        in_specs=[
            pl.BlockSpec((1, NUM_SEG, SEG_LEN, EMB),
                         lambda b: (b + SC_BATCH, 0, 0, 0)),
            pl.BlockSpec((NUM_SEG, EMB), lambda b: (0, 0)),
            pl.BlockSpec((SEG_LEN, EMB), lambda b: (0, 0)),
        ],
        out_specs=pl.BlockSpec((1, NUM_SEG, SEG_LEN, EMB),
                               lambda b: (b + SC_BATCH, 0, 0, 0)),
        out_shape=jax.ShapeDtypeStruct(
            (BATCH, NUM_SEG, SEG_LEN, EMB), x.dtype),
        compiler_params=pltpu.CompilerParams(
            dimension_semantics=("arbitrary",),
        ),
    )(x, seg_table, pos_table)


def kernel(x, seg_table, pos_table):
    x3 = x.reshape(BATCH * NUM_SEG, SEG_LEN, EMB)
    sc_out = _sc_run(x3, seg_table, pos_table)
    tc_out = _tc_run(x, seg_table, pos_table)
    return lax.dynamic_update_slice(
        tc_out, sc_out.reshape(SC_BATCH, NUM_SEG, SEG_LEN, EMB), (0, 0, 0, 0))


# speedup vs baseline: 1.1289x; 1.0066x over previous
import functools

import jax
import jax.numpy as jnp
from jax import lax
from jax.experimental import pallas as pl
from jax.experimental.pallas import tpu as pltpu
from jax.experimental.pallas import tpu_sc as plsc

BATCH = 16
NUM_SEG = 64
SEG_LEN = 128
EMB = 128

NC = 2
NS = 16
NW = NC * NS
ECH = EMB // 16

SC_BATCH = 1
TC_BATCH = BATCH - SC_BATCH
NBUF = 2
RU = 2


def _sc_run(x3, seg_table, pos_table):
    slabs = SC_BATCH * NUM_SEG
    spw = slabs // NW
    mesh = plsc.VectorSubcoreMesh(core_axis_name="c", subcore_axis_name="s")

    @functools.partial(
        pl.kernel,
        out_type=jax.ShapeDtypeStruct((slabs, SEG_LEN, EMB), jnp.float32),
        mesh=mesh,
        scratch_types=[
            pltpu.VMEM((NUM_SEG, EMB), jnp.float32),
            pltpu.VMEM((SEG_LEN, EMB), jnp.float32),
            [pltpu.VMEM((SEG_LEN, EMB), jnp.float32)] * NBUF,
            [pltpu.SemaphoreType.DMA] * NBUF,
            [pltpu.SemaphoreType.DMA] * NBUF,
            pltpu.SemaphoreType.DMA,
        ],
    )
    def run(x_hbm, seg_hbm, pos_hbm, out_hbm, seg_v, pos_v, bufs, in_sems,
            out_sems, tab_sem):
        c = lax.axis_index("c")
        s = lax.axis_index("s")
        wid = s * NC + c
        base = wid * spw

        def start_in(i, b):
            return pltpu.async_copy(x_hbm.at[base + i], bufs[b], in_sems[b])

        seg_h = pltpu.async_copy(seg_hbm, seg_v, tab_sem)
        pos_h = pltpu.async_copy(pos_hbm, pos_v, tab_sem)

        in_h = {}
        out_h = {}
        for i in range(min(NBUF - 1, spw)):
            in_h[i] = start_in(i, i)
        seg_h.wait()
        pos_h.wait()

        for i in range(spw):
            b = i % NBUF
            in_h.pop(i).wait()
            slab = base + i
            n = lax.rem(slab, NUM_SEG)
            buf = bufs[b]
            segv = tuple(seg_v[n, pl.ds(e * 16, 16)] for e in range(ECH))

            def row(r, segc, buf=buf):
                for u in range(RU):
                    l = r * RU + u
                    for e in range(ECH):
                        xv = buf[l, pl.ds(e * 16, 16)]
                        pv = pos_v[l, pl.ds(e * 16, 16)]
                        buf[l, pl.ds(e * 16, 16)] = xv + pv + segc[e]
                return segc

            lax.fori_loop(0, SEG_LEN // RU, row, segv)
            out_h[i] = pltpu.async_copy(buf, out_hbm.at[slab], out_sems[b])

            nxt = i + NBUF - 1
            if nxt < spw:
                bn = nxt % NBUF
                prev = nxt - NBUF
                if prev >= 0:
                    out_h.pop(prev).wait()
                in_h[nxt] = start_in(nxt, bn)

        for i in sorted(out_h):
            out_h.pop(i).wait()

    return run(x3, seg_table, pos_table)


def _tc_body(x_ref, seg_ref, pos_ref, out_ref):
    x = x_ref[...]
    seg = seg_ref[...]
    pos = pos_ref[...]
    out_ref[...] = x + seg[None, :, None, :] + pos[None, None, :, :]


def _tc_run(x, seg_table, pos_table):
    return pl.pallas_call(
        _tc_body,
        grid=(TC_BATCH,),
        in_specs=[
            pl.BlockSpec((1, NUM_SEG, SEG_LEN, EMB),
                         lambda b: (b + SC_BATCH, 0, 0, 0)),
            pl.BlockSpec((NUM_SEG, EMB), lambda b: (0, 0)),
            pl.BlockSpec((SEG_LEN, EMB), lambda b: (0, 0)),
        ],
        out_specs=pl.BlockSpec((1, NUM_SEG, SEG_LEN, EMB),
                               lambda b: (b + SC_BATCH, 0, 0, 0)),
        out_shape=jax.ShapeDtypeStruct(
            (BATCH, NUM_SEG, SEG_LEN, EMB), x.dtype),
        compiler_params=pltpu.CompilerParams(
            dimension_semantics=("arbitrary",),
        ),
    )(x, seg_table, pos_table)


def kernel(x, seg_table, pos_table):
    x3 = x.reshape(BATCH * NUM_SEG, SEG_LEN, EMB)
    tc_out = _tc_run(x, seg_table, pos_table)
    sc_out = _sc_run(x3, seg_table, pos_table)
    return lax.dynamic_update_slice(
        tc_out, sc_out.reshape(SC_BATCH, NUM_SEG, SEG_LEN, EMB), (0, 0, 0, 0))
